# baseline (speedup 1.0000x reference)
import functools

import jax
import jax.numpy as jnp
from jax import lax
from jax.experimental import pallas as pl
from jax.experimental.pallas import tpu as pltpu
from jax.experimental.pallas import tpu_sc as plsc

_H = 14
_W = 14
_S = _H * _W
_B = 64
_C = 768
_CG = 128
_L = 16
_NW = 32
_NCG = _C // _CG
_UNITS = _B * _NCG
_UPW = _UNITS // _NW


def _sc_body(x_hbm, tp_hbm, masked_hbm, tmpl_hbm, tp_v, x_v, m_v, t_v):
    wid = lax.axis_index("s") * 2 + lax.axis_index("c")
    pltpu.sync_copy(tp_hbm, tp_v)

    def unit_body(u, _):
        uid = u * _NW + wid
        b = uid // _NCG
        c0 = (uid % _NCG) * _CG
        pltpu.sync_copy(x_hbm.at[b, :, pl.ds(c0, _CG)], x_v)

        for g in range(_CG // _L):
            gl = g * _L

            def amax_body(s, carry):
                mx, am = carry
                v = x_v[s, pl.ds(gl, _L)]
                better = v > mx
                return (jnp.where(better, v, mx),
                        jnp.where(better, s, am))

            mx0 = jnp.full((_L,), -jnp.inf, jnp.float32)
            am0 = jnp.zeros((_L,), jnp.int32)
            _, am = lax.fori_loop(0, _S, amax_body, (mx0, am0))
            base = am * _S

            def out_body(s, _):
                tv = plsc.load_gather(tp_v, [base + s])
                xv = x_v[s, pl.ds(gl, _L)]
                t_v[s, pl.ds(gl, _L)] = tv
                m_v[s, pl.ds(gl, _L)] = jnp.maximum(xv * tv, 0.0)
                return 0

            lax.fori_loop(0, _S, out_body, 0)

        pltpu.sync_copy(t_v, tmpl_hbm.at[b, :, pl.ds(c0, _CG)])
        pltpu.sync_copy(m_v, masked_hbm.at[b, :, pl.ds(c0, _CG)])
        return 0

    lax.fori_loop(0, _UPW, unit_body, 0)


def kernel(input, t_p):
    x = input
    b, h, w, c = x.shape
    s = h * w
    x3 = x.reshape(b, s, c)
    tp1 = t_p.reshape(s * s)

    mesh = plsc.VectorSubcoreMesh(core_axis_name="c", subcore_axis_name="s")
    run = functools.partial(
        pl.kernel,
        out_type=[
            jax.ShapeDtypeStruct((b, s, c), jnp.float32),
            jax.ShapeDtypeStruct((b, s, c), jnp.float32),
        ],
        mesh=mesh,
        compiler_params=pltpu.CompilerParams(needs_layout_passes=False),
        scratch_types=[
            pltpu.VMEM((s * s,), jnp.float32),
            pltpu.VMEM((s, _CG), jnp.float32),
            pltpu.VMEM((s, _CG), jnp.float32),
            pltpu.VMEM((s, _CG), jnp.float32),
        ],
    )(_sc_body)
    masked, tmpl = run(x3, tp1)
    return (masked.reshape(b, h, w, c), x, tmpl.reshape(b, h, w, c))


# speedup vs baseline: 1.7214x; 1.7214x over previous
import functools

import jax
import jax.numpy as jnp
from jax import lax
from jax.experimental import pallas as pl
from jax.experimental.pallas import tpu as pltpu
from jax.experimental.pallas import tpu_sc as plsc

_H = 14
_W = 14
_S = _H * _W
_B = 64
_C = 768
_CG = 128
_L = 16
_NW = 32
_NCG = _C // _CG
_UNITS = _B * _NCG
_UPW = _UNITS // _NW


def _sc_body(x_hbm, tp_hbm, masked_hbm, tmpl_hbm, tp_v, x_v, m_v, t_v):
    wid = lax.axis_index("s") * 2 + lax.axis_index("c")
    pltpu.sync_copy(tp_hbm, tp_v)

    def unit_body(u, _):
        uid = u * _NW + wid
        b = uid // _NCG
        c0 = (uid % _NCG) * _CG
        pltpu.sync_copy(x_hbm.at[b, :, pl.ds(c0, _CG)], x_v)

        for g in range(_CG // _L):
            gl = g * _L

            mx0 = jnp.full((_L,), -jnp.inf, jnp.float32)
            am0 = jnp.zeros((_L,), jnp.int32)

            @plsc.parallel_loop(0, _S, 4, unroll=7, carry=(mx0, am0))
            def amax_loop(s, carry):
                mx, am = carry
                v0 = x_v[s, pl.ds(gl, _L)]
                v1 = x_v[s + 1, pl.ds(gl, _L)]
                v2 = x_v[s + 2, pl.ds(gl, _L)]
                v3 = x_v[s + 3, pl.ds(gl, _L)]
                b1 = v1 > v0
                m01 = jnp.where(b1, v1, v0)
                a01 = jnp.where(b1, s + 1, s)
                b3 = v3 > v2
                m23 = jnp.where(b3, v3, v2)
                a23 = jnp.where(b3, s + 3, s + 2)
                bb = m23 > m01
                ml = jnp.where(bb, m23, m01)
                al = jnp.where(bb, a23, a01)
                bc = ml > mx
                return (jnp.where(bc, ml, mx), jnp.where(bc, al, am))

            _, am = amax_loop
            base = am * _S

            @plsc.parallel_loop(0, _S, 2, unroll=7)
            def out_loop(s):
                for d in range(2):
                    tv = plsc.load_gather(tp_v, [base + (s + d)])
                    xv = x_v[s + d, pl.ds(gl, _L)]
                    t_v[s + d, pl.ds(gl, _L)] = tv
                    m_v[s + d, pl.ds(gl, _L)] = jnp.maximum(xv * tv, 0.0)

        pltpu.sync_copy(t_v, tmpl_hbm.at[b, :, pl.ds(c0, _CG)])
        pltpu.sync_copy(m_v, masked_hbm.at[b, :, pl.ds(c0, _CG)])
        return 0

    lax.fori_loop(0, _UPW, unit_body, 0)


def kernel(input, t_p):
    x = input
    b, h, w, c = x.shape
    s = h * w
    x3 = x.reshape(b, s, c)
    tp1 = t_p.reshape(s * s)

    mesh = plsc.VectorSubcoreMesh(core_axis_name="c", subcore_axis_name="s")
    run = functools.partial(
        pl.kernel,
        out_type=[
            jax.ShapeDtypeStruct((b, s, c), jnp.float32),
            jax.ShapeDtypeStruct((b, s, c), jnp.float32),
        ],
        mesh=mesh,
        compiler_params=pltpu.CompilerParams(needs_layout_passes=False),
        scratch_types=[
            pltpu.VMEM((s * s,), jnp.float32),
            pltpu.VMEM((s, _CG), jnp.float32),
            pltpu.VMEM((s, _CG), jnp.float32),
            pltpu.VMEM((s, _CG), jnp.float32),
        ],
    )(_sc_body)
    masked, tmpl = run(x3, tp1)
    return (masked.reshape(b, h, w, c), x, tmpl.reshape(b, h, w, c))
